# Initial kernel scaffold; baseline (speedup 1.0000x reference)
import jax
import jax.numpy as jnp
from jax.experimental import pallas as pl


def kernel(X, edge_index1, edge_weight1, edge_index2, edge_weight2, edge_index3, edge_weight3, gcn1_W1, gcn1_b1, gcn2_W1, gcn2_b1, gcn1_W2, gcn1_b2, gcn2_W2, gcn2_b2, gcn1_W3, gcn1_b3, gcn2_W3, gcn2_b3, lin1_W1, lin1_b1, lin1_W2, lin1_b2, lin1_W3, lin1_b3, lin2_W1, lin2_b1, lin2_W2, lin2_b2, lin2_W3, lin2_b3):
    raise NotImplementedError("write your pallas kernel here")


# speedup vs baseline: 8.2948x; 8.2948x over previous
import jax
import jax.numpy as jnp
from jax import lax
from jax.experimental import pallas as pl
from jax.experimental.pallas import tpu as pltpu
from jax.experimental.pallas import tpu_sc as plsc

N = 10000
E = 320000
D = 128
NC = 2
NS = 16
NW = NC * NS
K = 128
ECP = 10240
NCHUNK = ECP // K
NPAD = 10240
RPT = NPAD // NS
DEG_LEN = 3 * NPAD
DPT = DEG_LEN // NS
R = 1000


def _prep_edges(edge_index, edge_weight, goff):
    pad = NW * ECP - E
    zi = jnp.zeros((pad,), jnp.int32)
    src = jnp.concatenate([edge_index[0], zi]).reshape(NW, NCHUNK, K)
    dstf = jnp.concatenate([edge_index[1], zi])
    dst = dstf.reshape(NW, NCHUNK, K)
    dstoff = (dstf + goff).reshape(NW, NCHUNK, K)
    ew = jnp.concatenate([edge_weight, jnp.zeros((pad,), jnp.float32)])
    return src, dst, dstoff, ew.reshape(NW, NCHUNK, K)


def _deg_body(dst_hbm, ew_hbm, out_hbm, dstv, ewv, zb, acc, sem):
    cid = lax.axis_index("c")
    sid = lax.axis_index("s")
    wid = sid * NC + cid
    zero = jnp.zeros((16,), jnp.float32)

    def zloop(i, carry):
        zb[pl.ds(pl.multiple_of(i * 16, 16), 16)] = zero
        return carry

    lax.fori_loop(0, DPT // 16, zloop, 0)
    pltpu.sync_copy(zb, acc.at[pl.ds(sid * DPT, DPT)])
    pltpu.sync_copy(dst_hbm.at[wid], dstv)
    pltpu.sync_copy(ew_hbm.at[wid], ewv)
    plsc.subcore_barrier()

    def floop(s, carry):
        for q in range(8):
            c = s * 8 + q
            pltpu.async_copy(ewv.at[c], acc.at[dstv.at[c]], sem, add=True)
        for q in range(8):
            c = s * 8 + q
            pltpu.make_async_copy(ewv.at[c], acc.at[dstv.at[c]], sem).wait()
        return carry

    lax.fori_loop(0, (3 * NCHUNK) // 8, floop, 0)
    plsc.subcore_barrier()
    pltpu.sync_copy(acc.at[pl.ds(sid * DPT, DPT)],
                    out_hbm.at[cid, pl.ds(sid * DPT, DPT)])


_SC_CACHE = {}


def _deg_call(*args):
    if "deg" not in _SC_CACHE:
        _SC_CACHE["deg"] = pl.kernel(
            _deg_body,
            out_type=jax.ShapeDtypeStruct((NC, DEG_LEN), jnp.float32),
            mesh=plsc.VectorSubcoreMesh(core_axis_name="c",
                                        subcore_axis_name="s",
                                        num_cores=NC, num_subcores=NS),
            scratch_types=[
                pltpu.VMEM((3 * NCHUNK, K), jnp.int32),
                pltpu.VMEM((3 * NCHUNK, K), jnp.float32),
                pltpu.VMEM((DPT,), jnp.float32),
                pltpu.VMEM_SHARED((DEG_LEN,), jnp.float32),
                pltpu.SemaphoreType.DMA,
            ],
        )
    return _SC_CACHE["deg"](*args)


def _lane_bcast(vec, jj):
    idx = jnp.full((16, 1), jj, jnp.int32)
    return lax.gather(
        vec, idx,
        lax.GatherDimensionNumbers(offset_dims=(), collapsed_slice_dims=(0,),
                                   start_index_map=(0,)),
        (1,), mode=lax.GatherScatterMode.PROMISE_IN_BOUNDS)


def _seg_body(y_hbm, src_hbm, dst_hbm, ew_hbm, out_hbm,
              sbuf, dbuf, ebuf, rbuf, acc, isem0, isem1, gsem0, gsem1):
    cid = lax.axis_index("c")
    sid = lax.axis_index("s")
    wid = sid * NC + cid
    zero = jnp.zeros((16,), jnp.float32)
    isem = (isem0, isem1)
    gsem = (gsem0, gsem1)

    def start_idx(c, b):
        pltpu.async_copy(src_hbm.at[wid, c], sbuf.at[b], isem[b])
        pltpu.async_copy(dst_hbm.at[wid, c], dbuf.at[b], isem[b])
        pltpu.async_copy(ew_hbm.at[wid, c], ebuf.at[b], isem[b])

    def wait_idx(c, b):
        pltpu.make_async_copy(src_hbm.at[wid, c], sbuf.at[b], isem[b]).wait()
        pltpu.make_async_copy(dst_hbm.at[wid, c], dbuf.at[b], isem[b]).wait()
        pltpu.make_async_copy(ew_hbm.at[wid, c], ebuf.at[b], isem[b]).wait()

    def zrow(j, carry):
        for t in range(8):
            rbuf[0, j, pl.ds(16 * t, 16)] = zero
        return carry

    lax.fori_loop(0, K, zrow, 0)
    base = sid * RPT
    for b in range(RPT // K):
        pltpu.sync_copy(rbuf.at[0], acc.at[pl.ds(base + b * K, K)])
    plsc.subcore_barrier()

    start_idx(0, 0)
    start_idx(1, 1)
    wait_idx(0, 0)
    pltpu.async_copy(y_hbm.at[sbuf.at[0]], rbuf.at[0], gsem0)

    def pair(c2, carry):
        for b in range(2):
            c = 2 * c2 + b
            nb = 1 - b

            @pl.when(c < NCHUNK - 1)
            def _launch_next_gather():
                wait_idx(c + 1, nb)
                pltpu.async_copy(y_hbm.at[sbuf.at[nb]], rbuf.at[nb],
                                 gsem[nb])

            pltpu.make_async_copy(y_hbm.at[sbuf.at[b]], rbuf.at[b],
                                  gsem[b]).wait()

            def sgrp(q2, icarry):
                ew16 = ebuf[b, pl.ds(q2 * 16, 16)]
                for jj in range(16):
                    j = q2 * 16 + jj
                    w = _lane_bcast(ew16, jj)
                    for t in range(8):
                        sl = pl.ds(16 * t, 16)
                        rbuf[b, j, sl] = rbuf[b, j, sl] * w
                return icarry

            lax.fori_loop(0, K // 16, sgrp, 0)
            pltpu.sync_copy(rbuf.at[b], acc.at[dbuf.at[b]], add=True)

            @pl.when(c < NCHUNK - 2)
            def _start_next_idx():
                start_idx(c + 2, b)

        return carry

    lax.fori_loop(0, NCHUNK // 2, pair, 0)
    plsc.subcore_barrier()
    for b in range(RPT // K):
        pltpu.sync_copy(acc.at[pl.ds(base + b * K, K)],
                        out_hbm.at[cid, pl.ds(base + b * K, K)])


def _seg_call(*args):
    if "seg" not in _SC_CACHE:
        _SC_CACHE["seg"] = pl.kernel(
            _seg_body,
            out_type=jax.ShapeDtypeStruct((NC, NPAD, D), jnp.float32),
            mesh=plsc.VectorSubcoreMesh(core_axis_name="c",
                                        subcore_axis_name="s",
                                        num_cores=NC, num_subcores=NS),
            scratch_types=[
                pltpu.VMEM((2, K), jnp.int32),
                pltpu.VMEM((2, K), jnp.int32),
                pltpu.VMEM((2, K), jnp.float32),
                pltpu.VMEM((2, K, D), jnp.float32),
                pltpu.VMEM_SHARED((NPAD, D), jnp.float32),
                pltpu.SemaphoreType.DMA,
                pltpu.SemaphoreType.DMA,
                pltpu.SemaphoreType.DMA,
                pltpu.SemaphoreType.DMA,
            ],
        )
    return _SC_CACHE["seg"](*args)


def _full(shape):
    return pl.BlockSpec(shape, lambda i: tuple(0 for _ in shape))


def _dinv_body(d_ref, o_ref):
    o_ref[...] = lax.rsqrt(d_ref[0] + d_ref[1] + 1.0)


def _finish_deg(degacc):
    return pl.pallas_call(
        _dinv_body,
        out_shape=jax.ShapeDtypeStruct((3, NPAD), jnp.float32),
    )(degacc.reshape(NC, 3, NPAD))


def _fc_body(x_ref, W1, b1, W2, b2, W3, b3, o_ref):
    h = jnp.dot(x_ref[...], W1[...], preferred_element_type=jnp.float32)
    h = jnp.maximum(h + b1[...], 0.0)
    h = jnp.dot(h, W2[...], preferred_element_type=jnp.float32)
    h = jnp.maximum(h + b2[...], 0.0)
    h = jnp.dot(h, W3[...], preferred_element_type=jnp.float32)
    o_ref[...] = jnp.maximum(h + b3[...], 0.0)


def _mlp(x, W1, b1, W2, b2, W3, b3):
    return pl.pallas_call(
        _fc_body,
        grid=(N // R,),
        in_specs=[pl.BlockSpec((R, D), lambda i: (i, 0)),
                  _full((D, 256)), _full((1, 256)),
                  _full((256, D)), _full((1, D)),
                  _full((D, 64)), _full((1, 64))],
        out_specs=pl.BlockSpec((R, 64), lambda i: (i, 0)),
        out_shape=jax.ShapeDtypeStruct((N, 64), jnp.float32),
    )(x, W1, b1.reshape(1, -1), W2, b2.reshape(1, -1), W3, b3.reshape(1, -1))


def _pre_body(x_ref, W_ref, dv_ref, y_ref):
    y_ref[...] = jnp.dot(x_ref[...], W_ref[...],
                         preferred_element_type=jnp.float32) * dv_ref[...]


def _pre(x, W, dv):
    return pl.pallas_call(
        _pre_body,
        grid=(N // R,),
        in_specs=[pl.BlockSpec((R, D), lambda i: (i, 0)),
                  _full((D, D)),
                  pl.BlockSpec((R, 1), lambda i: (i, 0))],
        out_specs=pl.BlockSpec((R, D), lambda i: (i, 0)),
        out_shape=jax.ShapeDtypeStruct((N, D), jnp.float32),
    )(x, W, dv)


def _mid_body(acc_ref, y_ref, dv_ref, b_ref, W_ref, o_ref):
    a = acc_ref[0] + acc_ref[1] + y_ref[...]
    h = jnp.maximum(a * dv_ref[...] + b_ref[...], 0.0)
    o_ref[...] = jnp.dot(h, W_ref[...],
                         preferred_element_type=jnp.float32) * dv_ref[...]


def _mid(acc, y, dv, b, W):
    return pl.pallas_call(
        _mid_body,
        grid=(N // R,),
        in_specs=[pl.BlockSpec((NC, R, D), lambda i: (0, i, 0)),
                  pl.BlockSpec((R, D), lambda i: (i, 0)),
                  pl.BlockSpec((R, 1), lambda i: (i, 0)),
                  _full((1, D)),
                  _full((D, D))],
        out_specs=pl.BlockSpec((R, D), lambda i: (i, 0)),
        out_shape=jax.ShapeDtypeStruct((N, D), jnp.float32),
    )(acc, y, dv, b.reshape(1, -1), W)


def _post_body(acc_ref, y_ref, dv_ref, b_ref, W1, b1, W2, b2, W3, b3, o_ref):
    a = acc_ref[0] + acc_ref[1] + y_ref[...]
    h = jnp.maximum(a * dv_ref[...] + b_ref[...], 0.0)
    h = jnp.dot(h, W1[...], preferred_element_type=jnp.float32)
    h = jnp.maximum(h + b1[...], 0.0)
    h = jnp.dot(h, W2[...], preferred_element_type=jnp.float32)
    h = jnp.maximum(h + b2[...], 0.0)
    h = jnp.dot(h, W3[...], preferred_element_type=jnp.float32)
    o_ref[...] = jnp.maximum(h + b3[...], 0.0)


def _post(acc, y, dv, b, W1, b1, W2, b2, W3, b3):
    return pl.pallas_call(
        _post_body,
        grid=(N // R,),
        in_specs=[pl.BlockSpec((NC, R, D), lambda i: (0, i, 0)),
                  pl.BlockSpec((R, D), lambda i: (i, 0)),
                  pl.BlockSpec((R, 1), lambda i: (i, 0)),
                  _full((1, D)),
                  _full((D, 256)), _full((1, 256)),
                  _full((256, D)), _full((1, D)),
                  _full((D, 64)), _full((1, 64))],
        out_specs=pl.BlockSpec((R, 64), lambda i: (i, 0)),
        out_shape=jax.ShapeDtypeStruct((N, 64), jnp.float32),
    )(acc, y, dv, b.reshape(1, -1), W1, b1.reshape(1, -1),
      W2, b2.reshape(1, -1), W3, b3.reshape(1, -1))


def _add4_body(a_ref, b_ref, c_ref, d_ref, o_ref):
    o_ref[...] = a_ref[...] + b_ref[...] + c_ref[...] + d_ref[...]


def _add4(a, b, c, d):
    spec = pl.BlockSpec((R, 64), lambda i: (i, 0))
    return pl.pallas_call(
        _add4_body,
        grid=(N // R,),
        in_specs=[spec] * 4,
        out_specs=spec,
        out_shape=jax.ShapeDtypeStruct((N, 64), jnp.float32),
    )(a, b, c, d)


def kernel(X, edge_index1, edge_weight1, edge_index2, edge_weight2,
           edge_index3, edge_weight3,
           gcn1_W1, gcn1_b1, gcn2_W1, gcn2_b1,
           gcn1_W2, gcn1_b2, gcn2_W2, gcn2_b2,
           gcn1_W3, gcn1_b3, gcn2_W3, gcn2_b3,
           lin1_W1, lin1_b1, lin1_W2, lin1_b2, lin1_W3, lin1_b3,
           lin2_W1, lin2_b1, lin2_W2, lin2_b2, lin2_W3, lin2_b3):
    graphs = []
    doffs, ewds = [], []
    for g, (ei, ew) in enumerate([(edge_index1, edge_weight1),
                                  (edge_index2, edge_weight2),
                                  (edge_index3, edge_weight3)]):
        src, dst, dstoff, eww = _prep_edges(ei, ew, g * NPAD)
        graphs.append((src, dst, eww))
        doffs.append(dstoff)
        ewds.append(eww)

    degacc = _deg_call(jnp.concatenate(doffs, axis=1),
                       jnp.concatenate(ewds, axis=1))
    dinv_all = _finish_deg(degacc)

    X0 = _mlp(X, lin1_W1, lin1_b1, lin1_W2, lin1_b2, lin1_W3, lin1_b3)

    gcnW = [(gcn1_W1, gcn1_b1, gcn2_W1, gcn2_b1),
            (gcn1_W2, gcn1_b2, gcn2_W2, gcn2_b2),
            (gcn1_W3, gcn1_b3, gcn2_W3, gcn2_b3)]
    outs = []
    for g in range(3):
        src, dst, eww = graphs[g]
        Wa, ba, Wb, bb = gcnW[g]
        dv = dinv_all[g, :N].reshape(N, 1)
        y1 = _pre(X, Wa, dv)
        acc1 = _seg_call(y1, src, dst, eww)
        y2 = _mid(acc1, y1, dv, ba, Wb)
        acc2 = _seg_call(y2, src, dst, eww)
        o = _post(acc2, y2, dv, bb,
                  lin2_W1, lin2_b1, lin2_W2, lin2_b2, lin2_W3, lin2_b3)
        outs.append(o)

    Xout = _add4(X0, outs[0], outs[1], outs[2])
    return (Xout, outs[0], outs[1], outs[2])
